# baseline (speedup 1.0000x reference)
import functools

import jax
import jax.numpy as jnp
from jax.experimental import pallas as pl

TB = 2048


def _moe_block(x_ref, gw_ref, wgu_ref, wd_ref, o_ref):
    xb = x_ref[...]
    E = gw_ref.shape[0]
    F = wd_ref.shape[0] // E

    logits = jax.lax.dot_general(
        xb, gw_ref[...], (((1,), (1,)), ((), ())),
        preferred_element_type=jnp.float32)
    s = jax.nn.sigmoid(logits)
    eidx = jax.lax.broadcasted_iota(jnp.int32, s.shape, 1)

    m1 = jnp.max(s, axis=1, keepdims=True)
    i1 = jnp.min(jnp.where(s == m1, eidx, E), axis=1, keepdims=True)
    s2 = jnp.where(eidx == i1, -jnp.inf, s)
    m2 = jnp.max(s2, axis=1, keepdims=True)
    i2 = jnp.min(jnp.where(s2 == m2, eidx, E), axis=1, keepdims=True)
    denom = m1 + m2 + 1e-20
    combine = (jnp.where(eidx == i1, m1, 0.0)
               + jnp.where(eidx == i2, m2, 0.0)) / denom

    xb16 = xb.astype(jnp.bfloat16)
    hs = []
    for e in range(E):
        gu = jax.lax.dot_general(
            xb16, wgu_ref[e], (((1,), (0,)), ((), ())),
            preferred_element_type=jnp.float32)
        g = gu[:, :F]
        u = gu[:, F:]
        hs.append((jax.nn.silu(g) * u
                   * combine[:, e:e + 1]).astype(jnp.bfloat16))
    h = jnp.concatenate(hs, axis=1)
    o_ref[...] = jax.lax.dot_general(
        h, wd_ref[...], (((1,), (0,)), ((), ())),
        preferred_element_type=jnp.float32)


@jax.jit
def kernel(hidden_states, gate_weight, Wg, Wu, Wd):
    b, s, h = hidden_states.shape
    x = hidden_states.reshape(-1, h)
    T = x.shape[0]
    e, _, f = Wg.shape
    wgu = jnp.concatenate([Wg.astype(jnp.bfloat16), Wu.astype(jnp.bfloat16)],
                          axis=2)
    wd = Wd.reshape(e * f, h).astype(jnp.bfloat16)
    grid = (T // TB,)
    out = pl.pallas_call(
        _moe_block,
        grid=grid,
        in_specs=[
            pl.BlockSpec((TB, h), lambda i: (i, 0)),
            pl.BlockSpec((e, h), lambda i: (0, 0)),
            pl.BlockSpec((e, h, 2 * f), lambda i: (0, 0, 0)),
            pl.BlockSpec((e * f, h), lambda i: (0, 0)),
        ],
        out_specs=pl.BlockSpec((TB, h), lambda i: (i, 0)),
        out_shape=jax.ShapeDtypeStruct((T, h), jnp.float32),
    )(x, gate_weight, wgu, wd)
    return out.reshape(b, s, h)


# speedup vs baseline: 1.0572x; 1.0572x over previous
import functools

import jax
import jax.numpy as jnp
from jax.experimental import pallas as pl

TB = 1024


def _moe_block(x_ref, gw_ref, wgu_ref, wd_ref, o_ref):
    xb = x_ref[...]
    E = gw_ref.shape[0]
    F = wd_ref.shape[0] // E

    logits = jax.lax.dot_general(
        xb, gw_ref[...], (((1,), (1,)), ((), ())),
        preferred_element_type=jnp.float32)
    s = jax.nn.sigmoid(logits)
    eidx = jax.lax.broadcasted_iota(jnp.int32, s.shape, 1)

    m1 = jnp.max(s, axis=1, keepdims=True)
    i1 = jnp.min(jnp.where(s == m1, eidx, E), axis=1, keepdims=True)
    s2 = jnp.where(eidx == i1, -jnp.inf, s)
    m2 = jnp.max(s2, axis=1, keepdims=True)
    i2 = jnp.min(jnp.where(s2 == m2, eidx, E), axis=1, keepdims=True)
    denom = m1 + m2 + 1e-20
    combine = (jnp.where(eidx == i1, m1, 0.0)
               + jnp.where(eidx == i2, m2, 0.0)) / denom

    xb16 = xb.astype(jnp.bfloat16)
    hs = []
    for e in range(E):
        gu = jax.lax.dot_general(
            xb16, wgu_ref[e], (((1,), (0,)), ((), ())),
            preferred_element_type=jnp.float32)
        g = gu[:, :F]
        u = gu[:, F:]
        hs.append((jax.nn.silu(g) * u
                   * combine[:, e:e + 1]).astype(jnp.bfloat16))
    h = jnp.concatenate(hs, axis=1)
    o_ref[...] = jax.lax.dot_general(
        h, wd_ref[...], (((1,), (0,)), ((), ())),
        preferred_element_type=jnp.float32)


@jax.jit
def kernel(hidden_states, gate_weight, Wg, Wu, Wd):
    b, s, h = hidden_states.shape
    x = hidden_states.reshape(-1, h)
    T = x.shape[0]
    e, _, f = Wg.shape
    wgu = jnp.concatenate([Wg.astype(jnp.bfloat16), Wu.astype(jnp.bfloat16)],
                          axis=2)
    wd = Wd.reshape(e * f, h).astype(jnp.bfloat16)
    grid = (T // TB,)
    out = pl.pallas_call(
        _moe_block,
        grid=grid,
        in_specs=[
            pl.BlockSpec((TB, h), lambda i: (i, 0)),
            pl.BlockSpec((e, h), lambda i: (0, 0)),
            pl.BlockSpec((e, h, 2 * f), lambda i: (0, 0, 0)),
            pl.BlockSpec((e * f, h), lambda i: (0, 0)),
        ],
        out_specs=pl.BlockSpec((TB, h), lambda i: (i, 0)),
        out_shape=jax.ShapeDtypeStruct((T, h), jnp.float32),
    )(x, gate_weight, wgu, wd)
    return out.reshape(b, s, h)
